# baseline (speedup 1.0000x reference)
import functools

import jax
import jax.numpy as jnp
from jax import lax
from jax.experimental import pallas as pl
from jax.experimental.pallas import tpu as pltpu
from jax.experimental.pallas import tpu_sc as plsc

_N = 10000
_E = 320000
_F_IN = 128
_HID = 16
_LBL = 64

_NC = 2
_NS = 16
_NW = _NC * _NS
_CHUNK = 128
_G = 8
_EPW = 10240
_K = _EPW // (_CHUNK * _G)
_NBUF = 4
_DEPTH = 2
_NPAD = 10112
_RPT = _NPAD // _NS


def _edge_pass(gather: bool):
    mesh = plsc.VectorSubcoreMesh(core_axis_name="c", subcore_axis_name="s")

    @functools.partial(
        pl.kernel,
        mesh=mesh,
        compiler_params=pltpu.CompilerParams(use_tc_tiling_on_sc=False),
        out_type=jax.ShapeDtypeStruct((_NC, _NPAD, _HID), jnp.float32),
        scratch_types=[
            pltpu.VMEM((_K, _G * _CHUNK), jnp.int32),
            pltpu.VMEM((_K, _G * _CHUNK), jnp.int32),
            pltpu.VMEM((_NBUF, _G * _CHUNK, _HID), jnp.float32),
            pltpu.VMEM_SHARED((_NPAD, _HID), jnp.float32),
            pltpu.VMEM_SHARED((_NPAD, _HID), jnp.float32),
            pltpu.SemaphoreType.DMA((_NBUF,)),
            pltpu.SemaphoreType.DMA((_NBUF,)),
        ],
    )
    def k(src_hbm, dst_hbm, feat_hbm, zero_hbm, out_hbm,
          src_v, dst_v, rows_v, acc_sh, feat_sh, gsem, ssem):
        cid = lax.axis_index("c")
        sid = lax.axis_index("s")
        wid = sid * _NC + cid

        pltpu.sync_copy(zero_hbm.at[pl.ds(sid * _RPT, _RPT)],
                        acc_sh.at[pl.ds(sid * _RPT, _RPT)])
        pltpu.sync_copy(src_hbm.at[wid], src_v)
        pltpu.sync_copy(dst_hbm.at[wid], dst_v)
        if gather:
            pltpu.sync_copy(feat_hbm.at[pl.ds(sid * _RPT, _RPT)],
                            feat_sh.at[pl.ds(sid * _RPT, _RPT)])
        else:
            pltpu.sync_copy(feat_hbm.at[pl.ds(0, _G * _CHUNK)], rows_v.at[0])
        plsc.subcore_barrier()

        def gissue(j):
            s = lax.rem(j, _NBUF) if isinstance(j, jax.Array) else j % _NBUF
            pltpu.async_copy(feat_sh.at[src_v.at[j]], rows_v.at[s],
                             gsem.at[s])

        def sissue(j):
            s = lax.rem(j, _NBUF) if isinstance(j, jax.Array) else j % _NBUF
            if gather:
                pltpu.make_async_copy(feat_sh.at[src_v.at[j]], rows_v.at[s],
                                      gsem.at[s]).wait()
                pltpu.async_copy(rows_v.at[s], acc_sh.at[dst_v.at[j]],
                                 ssem.at[s], add=True)
            else:
                pltpu.async_copy(rows_v.at[0], acc_sh.at[dst_v.at[j]],
                                 ssem.at[s], add=True)

        def swait(j):
            s = lax.rem(j, _NBUF) if isinstance(j, jax.Array) else j % _NBUF
            pltpu.make_async_copy(rows_v.at[0], acc_sh.at[dst_v.at[0]],
                                  ssem.at[s]).wait()

        if gather:
            for j in range(_DEPTH):
                gissue(j)
            for j in range(_DEPTH, _NBUF):
                gissue(j)
                sissue(j - _DEPTH)

            def body(j, _):
                swait(j - _NBUF)
                gissue(j)
                sissue(j - _DEPTH)
                return 0
            lax.fori_loop(_NBUF, _K, body, 0)

            for j in range(_K - _DEPTH, _K):
                sissue(j)
            for j in range(_K - _NBUF, _K):
                swait(j)
        else:
            for j in range(_NBUF):
                sissue(j)

            def body(j, _):
                swait(j - _NBUF)
                sissue(j)
                return 0
            lax.fori_loop(_NBUF, _K, body, 0)
            for j in range(_K - _NBUF, _K):
                swait(j)

        plsc.subcore_barrier()
        pltpu.sync_copy(acc_sh.at[pl.ds(sid * _RPT, _RPT)],
                        out_hbm.at[cid, pl.ds(sid * _RPT, _RPT)])

    return k


_edge_pass = functools.lru_cache(maxsize=None)(_edge_pass)


def _matmul1_body(xp_ref, w1_ref, h_ref):
    h_ref[...] = jnp.dot(xp_ref[...], w1_ref[...],
                         preferred_element_type=jnp.float32)


def _scale1_body(h_ref, degp_ref, a1_ref, dv_ref):
    deg = degp_ref[0, :, 0:1] + degp_ref[1, :, 0:1] + 1.0
    dv = jnp.broadcast_to(lax.rsqrt(deg), (_NPAD, _HID))
    a1_ref[...] = dv * h_ref[...]
    dv_ref[...] = dv


def _dense2_body(p_ref, a1_ref, dv_ref, b1_ref, a2_ref):
    s = p_ref[0] + p_ref[1] + a1_ref[...]
    o = dv_ref[...] * s + b1_ref[...][None, :]
    a2_ref[...] = dv_ref[...] * jnp.maximum(o, 0.0)


def _dense3_body(p_ref, a2_ref, dv_ref, w2_ref, b2_ref, out_ref):
    pre = dv_ref[...] * (p_ref[0] + p_ref[1] + a2_ref[...])
    logits = jnp.dot(pre, w2_ref[...], preferred_element_type=jnp.float32)
    logits = logits + b2_ref[...][None, :]
    m = jnp.max(logits, axis=1, keepdims=True)
    s = logits - m
    out_ref[...] = s - jnp.log(jnp.sum(jnp.exp(s), axis=1, keepdims=True))


def kernel(x, edge_index, W1, b1, W2, b2):
    src = edge_index[0].astype(jnp.int32)
    dst = edge_index[1].astype(jnp.int32)
    pad = _NW * _EPW - _E
    padv = jnp.full((pad,), _N, jnp.int32)
    srcp = jnp.concatenate([src, padv]).reshape(_NW, _K, _G * _CHUNK)
    dstp = jnp.concatenate([dst, padv]).reshape(_NW, _K, _G * _CHUNK)
    zeros16 = jnp.zeros((_NPAD, _HID), jnp.float32)
    ones16 = jnp.ones((_NPAD, _HID), jnp.float32)
    xp = jnp.zeros((_NPAD, _F_IN), jnp.float32).at[:_N].set(x)

    h = pl.pallas_call(
        _matmul1_body,
        out_shape=jax.ShapeDtypeStruct((_NPAD, _HID), jnp.float32),
    )(xp, W1)

    degp = _edge_pass(False)(srcp, dstp, ones16, zeros16)

    a1, dv = pl.pallas_call(
        _scale1_body,
        out_shape=[jax.ShapeDtypeStruct((_NPAD, _HID), jnp.float32),
                   jax.ShapeDtypeStruct((_NPAD, _HID), jnp.float32)],
    )(h, degp)

    p1 = _edge_pass(True)(srcp, dstp, a1, zeros16)

    a2 = pl.pallas_call(
        _dense2_body,
        out_shape=jax.ShapeDtypeStruct((_NPAD, _HID), jnp.float32),
    )(p1, a1, dv, b1)

    p2 = _edge_pass(True)(srcp, dstp, a2, zeros16)

    out = pl.pallas_call(
        _dense3_body,
        out_shape=jax.ShapeDtypeStruct((_NPAD, _LBL), jnp.float32),
    )(p2, a2, dv, W2, b2)

    return out[:_N]


# speedup vs baseline: 1.0703x; 1.0703x over previous
import functools

import jax
import jax.numpy as jnp
from jax import lax
from jax.experimental import pallas as pl
from jax.experimental.pallas import tpu as pltpu
from jax.experimental.pallas import tpu_sc as plsc

_N = 10000
_E = 320000
_F_IN = 128
_HID = 16
_LBL = 64

_NC = 2
_NS = 16
_NW = _NC * _NS
_CHUNK = 128
_G = 8
_EPW = 10240
_K = _EPW // (_CHUNK * _G)
_NBUF = 4
_DEPTH = 2
_NPAD = 10112
_RPT = _NPAD // _NS


def _edge_pass(mode: str):
    mesh = plsc.VectorSubcoreMesh(core_axis_name="c", subcore_axis_name="s")

    out_main = jax.ShapeDtypeStruct((_NC, _NPAD, _HID), jnp.float32)
    tbl = jax.ShapeDtypeStruct((_NPAD, _HID), jnp.float32)
    if mode == "agg1":
        out_type = [out_main, tbl, tbl]
    elif mode == "agg2":
        out_type = [out_main, tbl]
    else:
        out_type = out_main

    scratch = [
        pltpu.VMEM((_K, _G * _CHUNK), jnp.int32),
        pltpu.VMEM((_K, _G * _CHUNK), jnp.int32),
        pltpu.VMEM((_NBUF, _G * _CHUNK, _HID), jnp.float32),
        pltpu.VMEM((_HID,), jnp.float32),
        pltpu.VMEM_SHARED((_NPAD, _HID), jnp.float32),
        pltpu.VMEM_SHARED((_NPAD, _HID), jnp.float32),
        pltpu.SemaphoreType.DMA((_NBUF,)),
        pltpu.SemaphoreType.DMA((_NBUF,)),
    ]

    def k(*refs):
        if mode == "agg1":
            (src_hbm, dst_hbm, h_hbm, degp_hbm, zero_hbm,
             out_hbm, a1_hbm, dv_hbm,
             src_v, dst_v, rows_v, b1_v,
             acc_sh, feat_sh, gsem, ssem) = refs
        elif mode == "agg2":
            (src_hbm, dst_hbm, a1t_hbm, dvt_hbm, p1_hbm, b1_hbm, zero_hbm,
             out_hbm, a2_hbm,
             src_v, dst_v, rows_v, b1_v,
             acc_sh, feat_sh, gsem, ssem) = refs
        else:
            (src_hbm, dst_hbm, feat_hbm, zero_hbm,
             out_hbm,
             src_v, dst_v, rows_v, b1_v,
             acc_sh, feat_sh, gsem, ssem) = refs
        gather = mode != "deg"
        cid = lax.axis_index("c")
        sid = lax.axis_index("s")
        wid = sid * _NC + cid
        r0 = sid * _RPT

        pltpu.sync_copy(zero_hbm.at[pl.ds(r0, _RPT)],
                        acc_sh.at[pl.ds(r0, _RPT)])
        pltpu.sync_copy(src_hbm.at[wid], src_v)
        pltpu.sync_copy(dst_hbm.at[wid], dst_v)

        if mode == "agg1":
            pltpu.sync_copy(h_hbm.at[pl.ds(r0, _RPT)],
                            rows_v.at[0, pl.ds(0, _RPT)])
            pltpu.sync_copy(degp_hbm.at[0, pl.ds(r0, _RPT)],
                            rows_v.at[1, pl.ds(0, _RPT)])
            pltpu.sync_copy(degp_hbm.at[1, pl.ds(r0, _RPT)],
                            rows_v.at[2, pl.ds(0, _RPT)])

            def prow1(j, _):
                d = rows_v[1, j] + rows_v[2, j] + 1.0
                i = lax.bitcast_convert_type(d, jnp.int32)
                i = jnp.int32(0x5F3759DF) - lax.shift_right_logical(i, 1)
                dvr = lax.bitcast_convert_type(i, jnp.float32)
                for _it in range(3):
                    dvr = dvr * (1.5 - 0.5 * d * dvr * dvr)
                rows_v[0, j] = dvr * rows_v[0, j]
                rows_v[1, j] = dvr
                return 0
            lax.fori_loop(0, _RPT, prow1, 0)
            pltpu.sync_copy(rows_v.at[0, pl.ds(0, _RPT)],
                            feat_sh.at[pl.ds(r0, _RPT)])

            @pl.when(cid == 0)
            def _():
                pltpu.sync_copy(rows_v.at[0, pl.ds(0, _RPT)],
                                a1_hbm.at[pl.ds(r0, _RPT)])
                pltpu.sync_copy(rows_v.at[1, pl.ds(0, _RPT)],
                                dv_hbm.at[pl.ds(r0, _RPT)])
        elif mode == "agg2":
            pltpu.sync_copy(a1t_hbm.at[pl.ds(r0, _RPT)],
                            rows_v.at[0, pl.ds(0, _RPT)])
            pltpu.sync_copy(dvt_hbm.at[pl.ds(r0, _RPT)],
                            rows_v.at[1, pl.ds(0, _RPT)])
            pltpu.sync_copy(p1_hbm.at[0, pl.ds(r0, _RPT)],
                            rows_v.at[2, pl.ds(0, _RPT)])
            pltpu.sync_copy(p1_hbm.at[1, pl.ds(r0, _RPT)],
                            rows_v.at[3, pl.ds(0, _RPT)])
            pltpu.sync_copy(b1_hbm, b1_v)

            def prow2(j, _):
                dvr = rows_v[1, j]
                pre = dvr * (rows_v[2, j] + rows_v[3, j] + rows_v[0, j])
                pre = pre + b1_v[...]
                rows_v[0, j] = dvr * jnp.maximum(pre, 0.0)
                return 0
            lax.fori_loop(0, _RPT, prow2, 0)
            pltpu.sync_copy(rows_v.at[0, pl.ds(0, _RPT)],
                            feat_sh.at[pl.ds(r0, _RPT)])

            @pl.when(cid == 0)
            def _():
                pltpu.sync_copy(rows_v.at[0, pl.ds(0, _RPT)],
                                a2_hbm.at[pl.ds(r0, _RPT)])
        else:
            pltpu.sync_copy(feat_hbm.at[pl.ds(0, _G * _CHUNK)], rows_v.at[0])
        plsc.subcore_barrier()

        def gissue(j):
            s = lax.rem(j, _NBUF) if isinstance(j, jax.Array) else j % _NBUF
            pltpu.async_copy(feat_sh.at[src_v.at[j]], rows_v.at[s],
                             gsem.at[s])

        def sissue(j):
            s = lax.rem(j, _NBUF) if isinstance(j, jax.Array) else j % _NBUF
            if gather:
                pltpu.make_async_copy(feat_sh.at[src_v.at[j]], rows_v.at[s],
                                      gsem.at[s]).wait()
                pltpu.async_copy(rows_v.at[s], acc_sh.at[dst_v.at[j]],
                                 ssem.at[s], add=True)
            else:
                pltpu.async_copy(rows_v.at[0], acc_sh.at[dst_v.at[j]],
                                 ssem.at[s], add=True)

        def swait(j):
            s = lax.rem(j, _NBUF) if isinstance(j, jax.Array) else j % _NBUF
            pltpu.make_async_copy(rows_v.at[0], acc_sh.at[dst_v.at[0]],
                                  ssem.at[s]).wait()

        if gather:
            for j in range(_DEPTH):
                gissue(j)
            for j in range(_DEPTH, _NBUF):
                gissue(j)
                sissue(j - _DEPTH)

            def body(j, _):
                swait(j - _NBUF)
                gissue(j)
                sissue(j - _DEPTH)
                return 0
            lax.fori_loop(_NBUF, _K, body, 0)

            for j in range(_K - _DEPTH, _K):
                sissue(j)
            for j in range(_K - _NBUF, _K):
                swait(j)
        else:
            for j in range(_NBUF):
                sissue(j)

            def body(j, _):
                swait(j - _NBUF)
                sissue(j)
                return 0
            lax.fori_loop(_NBUF, _K, body, 0)
            for j in range(_K - _NBUF, _K):
                swait(j)

        plsc.subcore_barrier()
        pltpu.sync_copy(acc_sh.at[pl.ds(sid * _RPT, _RPT)],
                        out_hbm.at[cid, pl.ds(sid * _RPT, _RPT)])

    return pl.kernel(
        k,
        mesh=mesh,
        compiler_params=pltpu.CompilerParams(use_tc_tiling_on_sc=False),
        out_type=out_type,
        scratch_types=scratch,
    )


_edge_pass = functools.lru_cache(maxsize=None)(_edge_pass)


def _matmul1_body(xp_ref, w1_ref, h_ref):
    h_ref[...] = jnp.dot(xp_ref[...], w1_ref[...],
                         preferred_element_type=jnp.float32)


def _dense3_body(p_ref, a2_ref, dv_ref, w2_ref, b2_ref, out_ref):
    pre = dv_ref[...] * (p_ref[0] + p_ref[1] + a2_ref[...])
    logits = jnp.dot(pre, w2_ref[...], preferred_element_type=jnp.float32)
    logits = logits + b2_ref[...][None, :]
    m = jnp.max(logits, axis=1, keepdims=True)
    s = logits - m
    out_ref[...] = s - jnp.log(jnp.sum(jnp.exp(s), axis=1, keepdims=True))


def kernel(x, edge_index, W1, b1, W2, b2):
    src = edge_index[0].astype(jnp.int32)
    dst = edge_index[1].astype(jnp.int32)
    pad = _NW * _EPW - _E
    padv = jnp.full((pad,), _N, jnp.int32)
    srcp = jnp.concatenate([src, padv]).reshape(_NW, _K, _G * _CHUNK)
    dstp = jnp.concatenate([dst, padv]).reshape(_NW, _K, _G * _CHUNK)
    zeros16 = jnp.zeros((_NPAD, _HID), jnp.float32)
    ones16 = jnp.ones((_NPAD, _HID), jnp.float32)
    xp = jnp.zeros((_NPAD, _F_IN), jnp.float32).at[:_N].set(x)

    h = pl.pallas_call(
        _matmul1_body,
        out_shape=jax.ShapeDtypeStruct((_NPAD, _HID), jnp.float32),
    )(xp, W1)

    degp = _edge_pass("deg")(srcp, dstp, ones16, zeros16)

    p1, a1, dv = _edge_pass("agg1")(srcp, dstp, h, degp, zeros16)

    p2, a2 = _edge_pass("agg2")(srcp, dstp, a1, dv, p1, b1, zeros16)

    out = pl.pallas_call(
        _dense3_body,
        out_shape=jax.ShapeDtypeStruct((_NPAD, _LBL), jnp.float32),
    )(p2, a2, dv, W2, b2)

    return out[:_N]


# speedup vs baseline: 1.2091x; 1.1297x over previous
import functools

import jax
import jax.numpy as jnp
from jax import lax
from jax.experimental import pallas as pl
from jax.experimental.pallas import tpu as pltpu
from jax.experimental.pallas import tpu_sc as plsc

_N = 10000
_E = 320000
_F_IN = 128
_HID = 16
_LBL = 64

_NC = 2
_NS = 16
_NW = _NC * _NS
_CHUNK = 128
_G = 8
_EPW = 10240
_K = _EPW // (_CHUNK * _G)
_NBUF = 4
_DEPTH = 2
_NPAD = 10112
_RPT = _NPAD // _NS


def _edge_pass(mode: str):
    mesh = plsc.VectorSubcoreMesh(core_axis_name="c", subcore_axis_name="s")

    out_main = jax.ShapeDtypeStruct((_NC, _NPAD, _HID), jnp.float32)
    tbl = jax.ShapeDtypeStruct((_NPAD, _HID), jnp.float32)
    if mode == "agg1":
        out_type = [out_main, tbl, tbl]
    elif mode == "agg2":
        out_type = [out_main, tbl]
    else:
        out_type = out_main

    scratch = [
        pltpu.VMEM((_K, _G * _CHUNK), jnp.int32),
        pltpu.VMEM((_K, _G * _CHUNK), jnp.int32),
        pltpu.VMEM((_NBUF, _G * _CHUNK, _HID), jnp.float32),
        pltpu.VMEM((_HID,), jnp.float32),
        pltpu.VMEM_SHARED((_NPAD, _HID), jnp.float32),
        pltpu.VMEM_SHARED((_NPAD, _HID), jnp.float32),
        pltpu.SemaphoreType.DMA((_NBUF,)),
        pltpu.SemaphoreType.DMA((_NBUF,)),
    ]

    def k(*refs):
        if mode == "agg1":
            (src_hbm, dst_hbm, h_hbm, degp_hbm, zero_hbm,
             out_hbm, a1_hbm, dv_hbm,
             src_v, dst_v, rows_v, b1_v,
             acc_sh, feat_sh, gsem, ssem) = refs
        elif mode == "agg2":
            (src_hbm, dst_hbm, a1t_hbm, dvt_hbm, p1_hbm, b1_hbm, zero_hbm,
             out_hbm, a2_hbm,
             src_v, dst_v, rows_v, b1_v,
             acc_sh, feat_sh, gsem, ssem) = refs
        else:
            (src_hbm, dst_hbm, feat_hbm, zero_hbm,
             out_hbm,
             src_v, dst_v, rows_v, b1_v,
             acc_sh, feat_sh, gsem, ssem) = refs
        gather = mode != "deg"
        cid = lax.axis_index("c")
        sid = lax.axis_index("s")
        wid = sid * _NC + cid
        r0 = sid * _RPT

        def stage(copies):
            for i, (s, d) in enumerate(copies):
                pltpu.async_copy(s, d, gsem.at[i % _NBUF])
            for i, (s, d) in enumerate(copies):
                pltpu.make_async_copy(s, d, gsem.at[i % _NBUF]).wait()

        common = [
            (zero_hbm.at[pl.ds(r0, _RPT)], acc_sh.at[pl.ds(r0, _RPT)]),
            (src_hbm.at[wid], src_v),
            (dst_hbm.at[wid], dst_v),
        ]

        if mode == "agg1":
            stage(common + [
                (h_hbm.at[pl.ds(r0, _RPT)], rows_v.at[0, pl.ds(0, _RPT)]),
                (degp_hbm.at[0, pl.ds(r0, _RPT)],
                 rows_v.at[1, pl.ds(0, _RPT)]),
                (degp_hbm.at[1, pl.ds(r0, _RPT)],
                 rows_v.at[2, pl.ds(0, _RPT)]),
            ])

            def prow1(j, _):
                d = rows_v[1, j] + rows_v[2, j] + 1.0
                i = lax.bitcast_convert_type(d, jnp.int32)
                i = jnp.int32(0x5F3759DF) - lax.shift_right_logical(i, 1)
                dvr = lax.bitcast_convert_type(i, jnp.float32)
                for _it in range(3):
                    dvr = dvr * (1.5 - 0.5 * d * dvr * dvr)
                rows_v[0, j] = dvr * rows_v[0, j]
                rows_v[1, j] = dvr
                return 0
            lax.fori_loop(0, _RPT, prow1, 0, unroll=8)
            pltpu.async_copy(rows_v.at[0, pl.ds(0, _RPT)],
                             feat_sh.at[pl.ds(r0, _RPT)], ssem.at[0])

            @pl.when(cid == 0)
            def _():
                pltpu.async_copy(rows_v.at[0, pl.ds(0, _RPT)],
                                 a1_hbm.at[pl.ds(r0, _RPT)], ssem.at[1])
                pltpu.async_copy(rows_v.at[1, pl.ds(0, _RPT)],
                                 dv_hbm.at[pl.ds(r0, _RPT)], ssem.at[2])
                pltpu.make_async_copy(rows_v.at[0, pl.ds(0, _RPT)],
                                      a1_hbm.at[pl.ds(r0, _RPT)],
                                      ssem.at[1]).wait()
                pltpu.make_async_copy(rows_v.at[1, pl.ds(0, _RPT)],
                                      dv_hbm.at[pl.ds(r0, _RPT)],
                                      ssem.at[2]).wait()
            pltpu.make_async_copy(rows_v.at[0, pl.ds(0, _RPT)],
                                  feat_sh.at[pl.ds(r0, _RPT)],
                                  ssem.at[0]).wait()
        elif mode == "agg2":
            stage(common + [
                (a1t_hbm.at[pl.ds(r0, _RPT)], rows_v.at[0, pl.ds(0, _RPT)]),
                (dvt_hbm.at[pl.ds(r0, _RPT)], rows_v.at[1, pl.ds(0, _RPT)]),
                (p1_hbm.at[0, pl.ds(r0, _RPT)],
                 rows_v.at[2, pl.ds(0, _RPT)]),
                (p1_hbm.at[1, pl.ds(r0, _RPT)],
                 rows_v.at[3, pl.ds(0, _RPT)]),
                (b1_hbm, b1_v),
            ])

            def prow2(j, _):
                dvr = rows_v[1, j]
                pre = dvr * (rows_v[2, j] + rows_v[3, j] + rows_v[0, j])
                pre = pre + b1_v[...]
                rows_v[0, j] = dvr * jnp.maximum(pre, 0.0)
                return 0
            lax.fori_loop(0, _RPT, prow2, 0, unroll=8)
            pltpu.async_copy(rows_v.at[0, pl.ds(0, _RPT)],
                             feat_sh.at[pl.ds(r0, _RPT)], ssem.at[0])

            @pl.when(cid == 0)
            def _():
                pltpu.async_copy(rows_v.at[0, pl.ds(0, _RPT)],
                                 a2_hbm.at[pl.ds(r0, _RPT)], ssem.at[1])
                pltpu.make_async_copy(rows_v.at[0, pl.ds(0, _RPT)],
                                      a2_hbm.at[pl.ds(r0, _RPT)],
                                      ssem.at[1]).wait()
            pltpu.make_async_copy(rows_v.at[0, pl.ds(0, _RPT)],
                                  feat_sh.at[pl.ds(r0, _RPT)],
                                  ssem.at[0]).wait()
        else:
            stage(common +
                  [(feat_hbm.at[pl.ds(0, _G * _CHUNK)], rows_v.at[0])])
        plsc.subcore_barrier()

        def gissue(j):
            s = lax.rem(j, _NBUF) if isinstance(j, jax.Array) else j % _NBUF
            pltpu.async_copy(feat_sh.at[src_v.at[j]], rows_v.at[s],
                             gsem.at[s])

        def sissue(j):
            s = lax.rem(j, _NBUF) if isinstance(j, jax.Array) else j % _NBUF
            if gather:
                pltpu.make_async_copy(feat_sh.at[src_v.at[j]], rows_v.at[s],
                                      gsem.at[s]).wait()
                pltpu.async_copy(rows_v.at[s], acc_sh.at[dst_v.at[j]],
                                 ssem.at[s], add=True)
            else:
                pltpu.async_copy(rows_v.at[0], acc_sh.at[dst_v.at[j]],
                                 ssem.at[s], add=True)

        def swait(j):
            s = lax.rem(j, _NBUF) if isinstance(j, jax.Array) else j % _NBUF
            pltpu.make_async_copy(rows_v.at[0], acc_sh.at[dst_v.at[0]],
                                  ssem.at[s]).wait()

        if gather:
            for j in range(_DEPTH):
                gissue(j)
            for j in range(_DEPTH, _NBUF):
                gissue(j)
                sissue(j - _DEPTH)

            def body(j, _):
                swait(j - _NBUF)
                gissue(j)
                sissue(j - _DEPTH)
                return 0
            lax.fori_loop(_NBUF, _K, body, 0)

            for j in range(_K - _DEPTH, _K):
                sissue(j)
            for j in range(_K - _NBUF, _K):
                swait(j)
        else:
            for j in range(_NBUF):
                sissue(j)

            def body(j, _):
                swait(j - _NBUF)
                sissue(j)
                return 0
            lax.fori_loop(_NBUF, _K, body, 0)
            for j in range(_K - _NBUF, _K):
                swait(j)

        plsc.subcore_barrier()
        pltpu.sync_copy(acc_sh.at[pl.ds(sid * _RPT, _RPT)],
                        out_hbm.at[cid, pl.ds(sid * _RPT, _RPT)])

    return pl.kernel(
        k,
        mesh=mesh,
        compiler_params=pltpu.CompilerParams(use_tc_tiling_on_sc=False),
        out_type=out_type,
        scratch_types=scratch,
    )


_edge_pass = functools.lru_cache(maxsize=None)(_edge_pass)


def _matmul1_body(xp_ref, w1_ref, h_ref):
    h_ref[...] = jnp.dot(xp_ref[...], w1_ref[...],
                         preferred_element_type=jnp.float32)


def _dense3_body(p_ref, a2_ref, dv_ref, w2_ref, b2_ref, out_ref):
    pre = dv_ref[...] * (p_ref[0] + p_ref[1] + a2_ref[...])
    logits = jnp.dot(pre, w2_ref[...], preferred_element_type=jnp.float32)
    logits = logits + b2_ref[...][None, :]
    m = jnp.max(logits, axis=1, keepdims=True)
    s = logits - m
    out_ref[...] = s - jnp.log(jnp.sum(jnp.exp(s), axis=1, keepdims=True))


def kernel(x, edge_index, W1, b1, W2, b2):
    src = edge_index[0].astype(jnp.int32)
    dst = edge_index[1].astype(jnp.int32)
    pad = _NW * _EPW - _E
    padv = jnp.full((pad,), _N, jnp.int32)
    srcp = jnp.concatenate([src, padv]).reshape(_NW, _K, _G * _CHUNK)
    dstp = jnp.concatenate([dst, padv]).reshape(_NW, _K, _G * _CHUNK)
    zeros16 = jnp.zeros((_NPAD, _HID), jnp.float32)
    ones16 = jnp.ones((_NPAD, _HID), jnp.float32)
    xp = jnp.zeros((_NPAD, _F_IN), jnp.float32).at[:_N].set(x)

    h = pl.pallas_call(
        _matmul1_body,
        out_shape=jax.ShapeDtypeStruct((_NPAD, _HID), jnp.float32),
    )(xp, W1)

    degp = _edge_pass("deg")(srcp, dstp, ones16, zeros16)

    p1, a1, dv = _edge_pass("agg1")(srcp, dstp, h, degp, zeros16)

    p2, a2 = _edge_pass("agg2")(srcp, dstp, a1, dv, p1, b1, zeros16)

    out = pl.pallas_call(
        _dense3_body,
        out_shape=jax.ShapeDtypeStruct((_NPAD, _LBL), jnp.float32),
    )(p2, a2, dv, W2, b2)

    return out[:_N]
